# baseline (speedup 1.0000x reference)
import functools

import jax
import jax.numpy as jnp
from jax import lax
from jax.experimental import pallas as pl
from jax.experimental.pallas import tpu as pltpu
from jax.experimental.pallas import tpu_sc as plsc

NC = 2
NS = 16
NW = NC * NS
BB = 128
L = 16


@functools.partial(jax.jit, static_argnames=("b", "s", "d"))
def _emb_lookup(emb_weight, x, b, s, d):
    mesh = plsc.VectorSubcoreMesh(core_axis_name="c", subcore_axis_name="s")
    rt = d // 8

    @functools.partial(
        pl.kernel,
        mesh=mesh,
        compiler_params=pltpu.CompilerParams(
            use_tc_tiling_on_sc=False, needs_layout_passes=False),
        out_type=jax.ShapeDtypeStruct((s, rt, NW, 8, 128), jnp.float32),
        scratch_types=(
            [pltpu.VMEM((BB, s), jnp.int32)]
            + [pltpu.VMEM((BB,), jnp.int32) for _ in range(2)]
            + [pltpu.VMEM((BB, d), jnp.float32) for _ in range(2)]
            + [pltpu.VMEM((rt, 8, 128), jnp.float32) for _ in range(2)]
            + [pltpu.SemaphoreType.DMA for _ in range(4)]
        ),
    )
    def k(table_hbm, x_hbm, out_hbm, xb_v, i0, i1, r0, r1, t0, t1,
          g0, g1, o0, o1):
        idxb = (i0, i1)
        rows = (r0, r1)
        trb = (t0, t1)
        gsem = (g0, g1)
        osem = (o0, o1)
        wid = lax.axis_index("s") * NC + lax.axis_index("c")
        b0 = wid * BB
        pltpu.sync_copy(x_hbm.at[pl.ds(b0, BB)], xb_v)
        iota = lax.iota(jnp.int32, L)
        rowv = [iota + L * kk for kk in range(BB // L)]

        def extract_idx(sq, bi):
            colv = jnp.zeros((L,), jnp.int32) + sq
            for kk in range(BB // L):
                v = plsc.load_gather(xb_v, [rowv[kk], colv])
                idxb[bi][pl.ds(L * kk, L)] = v

        def gather_start(bi):
            pltpu.async_copy(table_hbm.at[idxb[bi]], rows[bi], gsem[bi])

        def gather_wait(bi):
            pltpu.make_async_copy(
                table_hbm.at[idxb[bi]], rows[bi], gsem[bi]).wait()

        def transpose(bi):
            @plsc.parallel_loop(0, d, unroll=2)
            def t_body(j):
                colv = jnp.zeros((L,), jnp.int32) + j
                vals = [plsc.load_gather(rows[bi], [rowv[kk], colv])
                        for kk in range(BB // L)]
                for kk in range(BB // L):
                    trb[bi][j // 8, j % 8, pl.ds(L * kk, L)] = vals[kk]

        def out_start(sq, bi):
            pltpu.async_copy(trb[bi], out_hbm.at[sq, :, wid], osem[bi])

        def out_wait(bi):
            pltpu.make_async_copy(trb[bi], out_hbm.at[0, :, wid],
                                  osem[bi]).wait()

        extract_idx(0, 0)
        gather_start(0)

        n_groups = s // 2

        def outer(t, carry):
            s0 = 2 * t
            extract_idx(s0 + 1, 1)
            gather_start(1)
            gather_wait(0)

            @pl.when(t > 0)
            def _():
                out_wait(0)
            transpose(0)
            out_start(s0, 0)

            @pl.when(t < n_groups - 1)
            def _():
                extract_idx(s0 + 2, 0)
                gather_start(0)
            gather_wait(1)

            @pl.when(t > 0)
            def _():
                out_wait(1)
            transpose(1)
            out_start(s0 + 1, 1)
            return carry

        lax.fori_loop(0, n_groups, outer, 0)
        out_wait(0)
        out_wait(1)

    return k(emb_weight, x)


def kernel(x, emb_weight):
    b, s = x.shape
    v, d = emb_weight.shape
    out5d = _emb_lookup(emb_weight, x.astype(jnp.int32), b, s, d)
    t = jnp.transpose(out5d, (2, 4, 0, 1, 3))
    return t.reshape(b, s, d)


# speedup vs baseline: 2.2355x; 2.2355x over previous
import functools

import jax
import jax.numpy as jnp
from jax import lax
from jax.experimental import pallas as pl
from jax.experimental.pallas import tpu as pltpu
from jax.experimental.pallas import tpu_sc as plsc

NC = 2
NS = 16
NW = NC * NS
BB = 128
L = 16


@functools.partial(jax.jit, static_argnames=("b", "s", "d"))
def _emb_lookup(emb_weight, x, b, s, d):
    mesh = plsc.VectorSubcoreMesh(core_axis_name="c", subcore_axis_name="s")
    rt = d // 8

    @functools.partial(
        pl.kernel,
        mesh=mesh,
        compiler_params=pltpu.CompilerParams(
            use_tc_tiling_on_sc=False, needs_layout_passes=False),
        out_type=jax.ShapeDtypeStruct((s, rt, NW, 8, 128), jnp.float32),
        scratch_types=(
            [pltpu.VMEM((BB, s), jnp.int32)]
            + [pltpu.VMEM((BB,), jnp.int32) for _ in range(2)]
            + [pltpu.VMEM((BB, d), jnp.float32) for _ in range(2)]
            + [pltpu.VMEM((rt, 8, 129), jnp.float32) for _ in range(2)]
            + [pltpu.SemaphoreType.DMA for _ in range(4)]
        ),
    )
    def k(table_hbm, x_hbm, out_hbm, xb_v, i0, i1, r0, r1, t0, t1,
          g0, g1, o0, o1):
        idxb = (i0, i1)
        rows = (r0, r1)
        trb = (t0, t1)
        gsem = (g0, g1)
        osem = (o0, o1)
        wid = lax.axis_index("s") * NC + lax.axis_index("c")
        b0 = wid * BB
        pltpu.sync_copy(x_hbm.at[pl.ds(b0, BB)], xb_v)
        iota = lax.iota(jnp.int32, L)
        rowv = [iota + L * kk for kk in range(BB // L)]

        def extract_idx(sq, bi):
            colv = jnp.zeros((L,), jnp.int32) + sq
            for kk in range(BB // L):
                v = plsc.load_gather(xb_v, [rowv[kk], colv])
                idxb[bi][pl.ds(L * kk, L)] = v

        def gather_start(bi):
            pltpu.async_copy(table_hbm.at[idxb[bi]], rows[bi], gsem[bi])

        def gather_wait(bi):
            pltpu.make_async_copy(
                table_hbm.at[idxb[bi]], rows[bi], gsem[bi]).wait()

        dq = [L * q + iota for q in range(d // L)]
        rtv = [v // 8 for v in dq]
        rv = [v % 8 for v in dq]

        def transpose(bi):
            @plsc.parallel_loop(0, BB, unroll=2)
            def t_body(c):
                cb = jnp.zeros((L,), jnp.int32) + c
                vals = [rows[bi][c, pl.ds(L * q, L)] for q in range(d // L)]
                for q in range(d // L):
                    plsc.store_scatter(trb[bi], [rtv[q], rv[q], cb], vals[q])

        def out_start(sq, bi):
            pltpu.async_copy(trb[bi].at[:, :, pl.ds(0, 128)],
                             out_hbm.at[sq, :, wid], osem[bi])

        def out_wait(bi):
            pltpu.make_async_copy(trb[bi].at[:, :, pl.ds(0, 128)],
                                  out_hbm.at[0, :, wid], osem[bi]).wait()

        extract_idx(0, 0)
        gather_start(0)

        n_groups = s // 2

        def outer(t, carry):
            s0 = 2 * t
            extract_idx(s0 + 1, 1)
            gather_start(1)
            gather_wait(0)

            @pl.when(t > 0)
            def _():
                out_wait(0)
            transpose(0)
            out_start(s0, 0)

            @pl.when(t < n_groups - 1)
            def _():
                extract_idx(s0 + 2, 0)
                gather_start(0)
            gather_wait(1)

            @pl.when(t > 0)
            def _():
                out_wait(1)
            transpose(1)
            out_start(s0 + 1, 1)
            return carry

        lax.fori_loop(0, n_groups, outer, 0)
        out_wait(0)
        out_wait(1)

    return k(emb_weight, x)


def kernel(x, emb_weight):
    b, s = x.shape
    v, d = emb_weight.shape
    out5d = _emb_lookup(emb_weight, x.astype(jnp.int32), b, s, d)
    t = jnp.transpose(out5d, (2, 4, 0, 1, 3))
    return t.reshape(b, s, d)


# speedup vs baseline: 2.2591x; 1.0106x over previous
import functools

import jax
import jax.numpy as jnp
from jax import lax
from jax.experimental import pallas as pl
from jax.experimental.pallas import tpu as pltpu
from jax.experimental.pallas import tpu_sc as plsc

NC = 2
NS = 16
NW = NC * NS
BB = 128
L = 16


@functools.partial(jax.jit, static_argnames=("b", "s", "d"))
def _emb_lookup(emb_weight, x, b, s, d):
    mesh = plsc.VectorSubcoreMesh(core_axis_name="c", subcore_axis_name="s")
    rt = d // 8
    nq = d // L

    @functools.partial(
        pl.kernel,
        mesh=mesh,
        compiler_params=pltpu.CompilerParams(
            use_tc_tiling_on_sc=False, needs_layout_passes=False),
        out_type=jax.ShapeDtypeStruct((s, rt, NW, 8, 128), jnp.float32),
        scratch_types=(
            [pltpu.VMEM((BB, s), jnp.int32),
             pltpu.VMEM((s, BB), jnp.int32)]
            + [pltpu.VMEM((BB, d), jnp.float32) for _ in range(2)]
            + [pltpu.VMEM((rt, 8, 129), jnp.float32) for _ in range(2)]
            + [pltpu.SemaphoreType.DMA for _ in range(4)]
        ),
    )
    def k(table_hbm, x_hbm, out_hbm, xb_v, idxall, r0, r1, t0, t1,
          g0, g1, o0, o1):
        rows = (r0, r1)
        trb = (t0, t1)
        gsem = (g0, g1)
        osem = (o0, o1)
        wid = lax.axis_index("s") * NC + lax.axis_index("c")
        b0 = wid * BB
        pltpu.sync_copy(x_hbm.at[pl.ds(b0, BB)], xb_v)
        iota = lax.iota(jnp.int32, L)
        rowv = [iota + L * kk for kk in range(BB // L)]

        @plsc.parallel_loop(0, s, unroll=2)
        def extract(sq):
            colv = jnp.zeros((L,), jnp.int32) + sq
            vals = [plsc.load_gather(xb_v, [rowv[kk], colv])
                    for kk in range(BB // L)]
            for kk in range(BB // L):
                idxall[sq, pl.ds(L * kk, L)] = vals[kk]

        def gather_start(sq, bi):
            pltpu.async_copy(table_hbm.at[idxall.at[sq]], rows[bi], gsem[bi])

        def gather_wait(bi):
            pltpu.make_async_copy(
                table_hbm.at[idxall.at[0]], rows[bi], gsem[bi]).wait()

        dq = [L * q + iota for q in range(nq)]
        rtv = [v // 8 for v in dq]
        rv = [v % 8 for v in dq]

        def transpose(bi):
            @plsc.parallel_loop(0, BB, unroll=2)
            def t_body(c):
                cb = jnp.zeros((L,), jnp.int32) + c
                vals = [rows[bi][c, pl.ds(L * q, L)] for q in range(nq)]
                for q in range(nq):
                    plsc.store_scatter(trb[bi], [rtv[q], rv[q], cb], vals[q])

        def out_start(sq, bi):
            pltpu.async_copy(trb[bi].at[:, :, pl.ds(0, 128)],
                             out_hbm.at[sq, :, wid], osem[bi])

        def out_wait(bi):
            pltpu.make_async_copy(trb[bi].at[:, :, pl.ds(0, 128)],
                                  out_hbm.at[0, :, wid], osem[bi]).wait()

        gather_start(0, 0)
        gather_start(1, 1)

        n_groups = s // 2

        def outer(t, carry):
            s0 = 2 * t
            gather_wait(0)

            @pl.when(t > 0)
            def _():
                out_wait(0)
            transpose(0)

            @pl.when(t < n_groups - 1)
            def _():
                gather_start(s0 + 2, 0)
            out_start(s0, 0)

            gather_wait(1)

            @pl.when(t > 0)
            def _():
                out_wait(1)
            transpose(1)

            @pl.when(t < n_groups - 1)
            def _():
                gather_start(s0 + 3, 1)
            out_start(s0 + 1, 1)
            return carry

        lax.fori_loop(0, n_groups, outer, 0)
        out_wait(0)
        out_wait(1)

    return k(emb_weight, x)


def kernel(x, emb_weight):
    b, s = x.shape
    v, d = emb_weight.shape
    out5d = _emb_lookup(emb_weight, x.astype(jnp.int32), b, s, d)
    t = jnp.transpose(out5d, (2, 4, 0, 1, 3))
    return t.reshape(b, s, d)


# speedup vs baseline: 2.2638x; 1.0020x over previous
import functools

import jax
import jax.numpy as jnp
from jax import lax
from jax.experimental import pallas as pl
from jax.experimental.pallas import tpu as pltpu
from jax.experimental.pallas import tpu_sc as plsc

NC = 2
NS = 16
NW = NC * NS
BB = 128
L = 16


@functools.partial(jax.jit, static_argnames=("b", "s", "d"))
def _emb_lookup(emb_weight, x, b, s, d):
    mesh = plsc.VectorSubcoreMesh(core_axis_name="c", subcore_axis_name="s")
    rt = d // 8
    nq = d // L

    @functools.partial(
        pl.kernel,
        mesh=mesh,
        compiler_params=pltpu.CompilerParams(
            use_tc_tiling_on_sc=False, needs_layout_passes=False),
        out_type=jax.ShapeDtypeStruct((s, rt, NW, 8, 128), jnp.float32),
        scratch_types=(
            [pltpu.VMEM((BB, s), jnp.int32),
             pltpu.VMEM((s, BB), jnp.int32)]
            + [pltpu.VMEM((BB, d), jnp.float32) for _ in range(2)]
            + [pltpu.VMEM((rt, 8, 129), jnp.float32) for _ in range(2)]
            + [pltpu.SemaphoreType.DMA for _ in range(4)]
        ),
    )
    def k(table_hbm, x_hbm, out_hbm, xb_v, idxall, r0, r1, t0, t1,
          g0, g1, o0, o1):
        rows = (r0, r1)
        trb = (t0, t1)
        gsem = (g0, g1)
        osem = (o0, o1)
        wid = lax.axis_index("s") * NC + lax.axis_index("c")
        b0 = wid * BB
        pltpu.sync_copy(x_hbm.at[pl.ds(b0, BB)], xb_v)
        iota = lax.iota(jnp.int32, L)
        rowv = [iota + L * kk for kk in range(BB // L)]

        @plsc.parallel_loop(0, s, unroll=2)
        def extract(sq):
            colv = jnp.zeros((L,), jnp.int32) + sq
            vals = [plsc.load_gather(xb_v, [rowv[kk], colv])
                    for kk in range(BB // L)]
            for kk in range(BB // L):
                idxall[sq, pl.ds(L * kk, L)] = vals[kk]

        def gather_start(sq, bi):
            pltpu.async_copy(table_hbm.at[idxall.at[sq]], rows[bi], gsem[bi])

        def gather_wait(bi):
            pltpu.make_async_copy(
                table_hbm.at[idxall.at[0]], rows[bi], gsem[bi]).wait()

        dq = [L * q + iota for q in range(nq)]
        rtv = [v // 8 for v in dq]
        rv = [v % 8 for v in dq]

        def transpose(bi):
            @plsc.parallel_loop(0, BB, unroll=4)
            def t_body(c):
                cb = jnp.zeros((L,), jnp.int32) + c
                vals = [rows[bi][c, pl.ds(L * q, L)] for q in range(nq)]
                for q in range(nq):
                    plsc.store_scatter(trb[bi], [rtv[q], rv[q], cb], vals[q])

        def out_start(sq, bi):
            pltpu.async_copy(trb[bi].at[:, :, pl.ds(0, 128)],
                             out_hbm.at[sq, :, wid], osem[bi])

        def out_wait(bi):
            pltpu.make_async_copy(trb[bi].at[:, :, pl.ds(0, 128)],
                                  out_hbm.at[0, :, wid], osem[bi]).wait()

        gather_start(0, 0)
        gather_start(1, 1)

        n_groups = s // 2

        def outer(t, carry):
            s0 = 2 * t
            gather_wait(0)

            @pl.when(t > 0)
            def _():
                out_wait(0)
            transpose(0)

            @pl.when(t < n_groups - 1)
            def _():
                gather_start(s0 + 2, 0)
            out_start(s0, 0)

            gather_wait(1)

            @pl.when(t > 0)
            def _():
                out_wait(1)
            transpose(1)

            @pl.when(t < n_groups - 1)
            def _():
                gather_start(s0 + 3, 1)
            out_start(s0 + 1, 1)
            return carry

        lax.fori_loop(0, n_groups, outer, 0)
        out_wait(0)
        out_wait(1)

    return k(emb_weight, x)


def kernel(x, emb_weight):
    b, s = x.shape
    v, d = emb_weight.shape
    out5d = _emb_lookup(emb_weight, x.astype(jnp.int32), b, s, d)
    t = jnp.transpose(out5d, (2, 4, 0, 1, 3))
    return t.reshape(b, s, d)


# speedup vs baseline: 2.3117x; 1.0212x over previous
import functools

import jax
import jax.numpy as jnp
from jax import lax
from jax.experimental import pallas as pl
from jax.experimental.pallas import tpu as pltpu
from jax.experimental.pallas import tpu_sc as plsc

NC = 2
NS = 16
NW = NC * NS
BB = 128
L = 16
SP = 2


@functools.partial(jax.jit, static_argnames=("b", "s", "d"))
def _emb_lookup(emb_weight, x, b, s, d):
    mesh = plsc.VectorSubcoreMesh(core_axis_name="c", subcore_axis_name="s")
    rt = d // 8
    nq = d // L
    nr = s // SP

    @functools.partial(
        pl.kernel,
        mesh=mesh,
        compiler_params=pltpu.CompilerParams(
            use_tc_tiling_on_sc=False, needs_layout_passes=False),
        out_type=jax.ShapeDtypeStruct((s, rt, NW, 8, 128), jnp.float32),
        scratch_types=(
            [pltpu.VMEM((BB, s), jnp.int32),
             pltpu.VMEM((nr, SP * BB), jnp.int32)]
            + [pltpu.VMEM((SP * BB, d), jnp.float32) for _ in range(2)]
            + [pltpu.VMEM((rt, 8, 129), jnp.float32) for _ in range(4)]
            + [pltpu.SemaphoreType.DMA for _ in range(6)]
        ),
    )
    def k(table_hbm, x_hbm, out_hbm, xb_v, idxall, r0, r1, t0, t1, t2, t3,
          g0, g1, o0, o1, o2, o3):
        rows = (r0, r1)
        trb = (t0, t1, t2, t3)
        gsem = (g0, g1)
        osem = (o0, o1, o2, o3)
        wid = lax.axis_index("s") * NC + lax.axis_index("c")
        b0 = wid * BB
        pltpu.sync_copy(x_hbm.at[pl.ds(b0, BB)], xb_v)
        iota = lax.iota(jnp.int32, L)
        rowv = [iota + L * kk for kk in range(BB // L)]

        @plsc.parallel_loop(0, s, unroll=2)
        def extract(sq):
            colv = jnp.zeros((L,), jnp.int32) + sq
            vals = [plsc.load_gather(xb_v, [rowv[kk], colv])
                    for kk in range(BB // L)]
            base = (sq % SP) * BB
            for kk in range(BB // L):
                idxall[sq // SP, pl.ds(base + L * kk, L)] = vals[kk]

        def gather_start(rnd, bi):
            pltpu.async_copy(table_hbm.at[idxall.at[rnd]], rows[bi], gsem[bi])

        def gather_wait(bi):
            pltpu.make_async_copy(
                table_hbm.at[idxall.at[0]], rows[bi], gsem[bi]).wait()

        dq = [L * q + iota for q in range(nq)]
        rtv = [v // 8 for v in dq]
        rv = [v % 8 for v in dq]

        def transpose(bi, half, ti):
            @plsc.parallel_loop(0, BB, unroll=4)
            def t_body(c):
                cb = jnp.zeros((L,), jnp.int32) + c
                cr = half * BB + c
                vals = [rows[bi][cr, pl.ds(L * q, L)] for q in range(nq)]
                for q in range(nq):
                    plsc.store_scatter(trb[ti], [rtv[q], rv[q], cb], vals[q])

        def out_start(sq, ti):
            pltpu.async_copy(trb[ti].at[:, :, pl.ds(0, 128)],
                             out_hbm.at[sq, :, wid], osem[ti])

        def out_wait(ti):
            pltpu.make_async_copy(trb[ti].at[:, :, pl.ds(0, 128)],
                                  out_hbm.at[0, :, wid], osem[ti]).wait()

        def do_round(rnd, bi, wait_out):
            gather_wait(bi)
            for half in range(SP):
                ti = 2 * bi + half
                if wait_out:
                    out_wait(ti)
                transpose(bi, half, ti)
                if half == 0:
                    if isinstance(rnd, int):
                        if rnd + 2 < nr:
                            gather_start(rnd + 2, bi)
                    else:
                        @pl.when(rnd + 2 < nr)
                        def _():
                            gather_start(rnd + 2, bi)
                out_start(SP * rnd + half, ti)

        gather_start(0, 0)
        gather_start(1, 1)

        def outer(g, carry):
            do_round(2 * g, 0, True)
            do_round(2 * g + 1, 1, True)
            return carry

        do_round(0, 0, False)
        do_round(1, 1, False)
        lax.fori_loop(1, (nr - 1) // 2, outer, 0)
        do_round(nr - 1, 0, True)
        for ti in range(4):
            out_wait(ti)

    return k(emb_weight, x)


def kernel(x, emb_weight):
    b, s = x.shape
    v, d = emb_weight.shape
    out5d = _emb_lookup(emb_weight, x.astype(jnp.int32), b, s, d)
    t = jnp.transpose(out5d, (2, 4, 0, 1, 3))
    return t.reshape(b, s, d)
